# baseline (speedup 1.0000x reference)
import jax
import jax.numpy as jnp
from jax import lax
from jax.experimental import pallas as pl
from jax.experimental.pallas import tpu as pltpu
from jax.experimental.pallas import tpu_sc as plsc

VOCAB = 1000000
D = 64
B = 16384
NC = 2
NS = 16
NW = NC * NS
BPW = B // NW
LANES = 16
RPW = 31248
TAIL = VOCAB - NW * RPW


def _repack_body(table_hbm, out_hbm, sem):
  wid = lax.axis_index("s") * NC + lax.axis_index("c")
  r0 = pl.multiple_of(wid * RPW, 8)
  pltpu.async_copy(table_hbm.at[pl.ds(r0, RPW), :],
                   out_hbm.at[pl.ds(r0, RPW), :], sem).wait()

  @pl.when(wid == 0)
  def _():
    t0 = pl.multiple_of(NW * RPW, 8)
    pltpu.async_copy(table_hbm.at[pl.ds(t0, TAIL), :],
                     out_hbm.at[pl.ds(t0, TAIL), :], sem).wait()


def _gather_body(table_hbm, idx1_hbm, idx2_hbm, vis1_hbm, vis2_hbm,
                 out1_hbm, out2_hbm, idx_v, rows_v, acc_v, sem):
  wid = lax.axis_index("s") * NC + lax.axis_index("c")
  base = wid * BPW

  def one_lookup(idx_hbm, vis_hbm, out_hbm):
    pltpu.sync_copy(idx_hbm.at[pl.ds(base, BPW)], idx_v)
    gather = pltpu.async_copy(table_hbm.at[idx_v], rows_v, sem)
    pltpu.sync_copy(vis_hbm.at[0, pl.ds(base * D, BPW * D)], acc_v)
    gather.wait()

    def add_row(r, carry):
      for j in range(D // LANES):
        fsl = pl.ds(r * D + j * LANES, LANES)
        acc_v[fsl] = acc_v[fsl] + rows_v[r, pl.ds(j * LANES, LANES)]
      return carry

    lax.fori_loop(0, BPW, add_row, 0)
    pltpu.sync_copy(acc_v, out_hbm.at[0, pl.ds(base * D, BPW * D)])

  one_lookup(idx1_hbm, vis1_hbm, out1_hbm)
  one_lookup(idx2_hbm, vis2_hbm, out2_hbm)


@jax.jit
def _run(vis1, vis2, idx1, idx2, table):
  mesh = plsc.VectorSubcoreMesh(
      core_axis_name="c", subcore_axis_name="s",
      num_cores=NC, num_subcores=NS)
  table_lin = pl.kernel(
      _repack_body,
      out_type=jax.ShapeDtypeStruct((VOCAB, D), jnp.float32),
      mesh=mesh,
      scratch_types=[pltpu.SemaphoreType.DMA],
      compiler_params=pltpu.CompilerParams(
          use_tc_tiling_on_sc=True, needs_layout_passes=False),
  )(table)
  return pl.kernel(
      _gather_body,
      out_type=(jax.ShapeDtypeStruct((1, B * D), jnp.float32),
                jax.ShapeDtypeStruct((1, B * D), jnp.float32)),
      mesh=mesh,
      scratch_types=[
          pltpu.VMEM((BPW,), jnp.int32),
          pltpu.VMEM((BPW, D), jnp.float32),
          pltpu.VMEM((BPW * D,), jnp.float32),
          pltpu.SemaphoreType.DMA,
      ],
      compiler_params=pltpu.CompilerParams(
          use_tc_tiling_on_sc=False, needs_layout_passes=False),
  )(table_lin, idx1, idx2, vis1, vis2)


def kernel(vis_features_first, vis_features_second, inputs_first,
           inputs_second, embedding_table):
  idx1 = inputs_first.astype(jnp.int32)
  idx2 = inputs_second.astype(jnp.int32)
  return _run(vis_features_first, vis_features_second, idx1, idx2,
              embedding_table)


# speedup vs baseline: 36.7824x; 36.7824x over previous
import jax
import jax.numpy as jnp
from jax import lax
from jax.experimental import pallas as pl
from jax.experimental.pallas import tpu as pltpu
from jax.experimental.pallas import tpu_sc as plsc

VOCAB = 1000000
D = 64
B = 16384
NC = 2
NS = 16
NW = NC * NS
BPW = B // NW
LANES = 16
CH = 16


def _sc_body(table_hbm, idx1_hbm, idx2_hbm, vis1_hbm, vis2_hbm,
             out1_hbm, out2_hbm, idx_v, grp_v, acc_v, sem):
  wid = lax.axis_index("s") * NC + lax.axis_index("c")
  base = wid * BPW
  lane = lax.iota(jnp.int32, LANES)

  def one_lookup(idx_hbm, vis_hbm, out_hbm):
    pltpu.sync_copy(idx_hbm.at[pl.ds(base, BPW)], idx_v)
    pltpu.sync_copy(vis_hbm.at[0, pl.ds(base * D, BPW * D)], acc_v)

    def fire(c0, boff):
      v16 = idx_v[pl.ds(c0, CH)]
      scalars = []
      for k in range(CH):
        vk = lax.reduce_sum(jnp.where(lane == k, v16, 0), axes=(0,))
        scalars.append(vk)
      copies = []
      for k in range(CH):
        g = pl.multiple_of((scalars[k] // 8) * 8, 8)
        copies.append(pltpu.async_copy(
            table_hbm.at[pl.ds(g, 8), :],
            grp_v.at[pl.ds(boff + k * 8, 8), :], sem))
      return copies, scalars

    def extract(c0, boff, scalars):
      for k in range(CH):
        row16 = jnp.full((LANES,), scalars[k] % 8 + boff + k * 8, jnp.int32)
        for j in range(D // LANES):
          col16 = lane + (j * LANES)
          fsl = pl.ds((c0 + k) * D + j * LANES, LANES)
          acc_v[fsl] = acc_v[fsl] + plsc.load_gather(grp_v, [row16, col16])

    def do_pair(p, carry):
      c0a = (2 * p) * CH
      c0b = (2 * p + 1) * CH
      copies_a, scal_a = fire(c0a, 0)
      copies_b, scal_b = fire(c0b, CH * 8)
      for cp in copies_a:
        cp.wait()
      extract(c0a, 0, scal_a)
      for cp in copies_b:
        cp.wait()
      extract(c0b, CH * 8, scal_b)
      return carry

    lax.fori_loop(0, BPW // (2 * CH), do_pair, 0)
    pltpu.sync_copy(acc_v, out_hbm.at[0, pl.ds(base * D, BPW * D)])

  one_lookup(idx1_hbm, vis1_hbm, out1_hbm)
  one_lookup(idx2_hbm, vis2_hbm, out2_hbm)


@jax.jit
def _run(vis1, vis2, idx1, idx2, table):
  mesh = plsc.VectorSubcoreMesh(
      core_axis_name="c", subcore_axis_name="s",
      num_cores=NC, num_subcores=NS)
  return pl.kernel(
      _sc_body,
      out_type=(jax.ShapeDtypeStruct((1, B * D), jnp.float32),
                jax.ShapeDtypeStruct((1, B * D), jnp.float32)),
      mesh=mesh,
      scratch_types=[
          pltpu.VMEM((BPW,), jnp.int32),
          pltpu.VMEM((2 * CH * 8, D), jnp.float32),
          pltpu.VMEM((BPW * D,), jnp.float32),
          pltpu.SemaphoreType.DMA,
      ],
      compiler_params=pltpu.CompilerParams(
          use_tc_tiling_on_sc=True, needs_layout_passes=False),
  )(table, idx1, idx2, vis1, vis2)


def kernel(vis_features_first, vis_features_second, inputs_first,
           inputs_second, embedding_table):
  idx1 = inputs_first.astype(jnp.int32)
  idx2 = inputs_second.astype(jnp.int32)
  return _run(vis_features_first, vis_features_second, idx1, idx2,
              embedding_table)
